# baseline (speedup 1.0000x reference)
import functools

import jax
import jax.numpy as jnp
from jax import lax
from jax.experimental import pallas as pl
from jax.experimental.pallas import tpu as pltpu
from jax.experimental.pallas import tpu_sc as plsc

B = 16384
N = 200
D = 128
V = 400
L = 16
K = 128
NBUF = 4

_info = plsc.get_sparse_core_info()
NC, NS = _info.num_cores, _info.num_subcores
NW = NC * NS
B_PER_W = B // NW
HB = 64
HG = B_PER_W // HB
ROWS_H = HB * N
STEPS = ROWS_H // K
SPR = 4

_mesh = plsc.VectorSubcoreMesh(core_axis_name="c", subcore_axis_name="s")


@functools.partial(
    pl.kernel,
    mesh=_mesh,
    out_type=jax.ShapeDtypeStruct((B * N, D), jnp.float32),
    scratch_types=[
        pltpu.VMEM((N * 2 * HB,), jnp.int32),
        pltpu.VMEM((HB * N,), jnp.int32),
        pltpu.VMEM((NBUF, K, D), jnp.float32),
        pltpu.SemaphoreType.DMA,
        (pltpu.SemaphoreType.DMA,) * NBUF,
        (pltpu.SemaphoreType.DMA,) * NBUF,
    ],
    compiler_params=pltpu.CompilerParams(needs_layout_passes=False),
)
def _sc_embed(xy_hbm, emb_hbm, out_hbm, xy_s, idxbuf, rbuf,
              sem_x, gsems, ssems):
    wid = lax.axis_index("s") * NC + lax.axis_index("c")
    iota = jnp.arange(L, dtype=jnp.int32)
    lane_base = iota * N

    def stage_one(h, o):
        bt = wid * 4 + h // 2
        bl0 = (h % 2) * HB
        n = o // 2
        c = o % 2
        pltpu.async_copy(
            xy_hbm.at[pl.ds(n * (128 * 256) + bt * 256 + c * 128 + bl0, HB)],
            xy_s.at[pl.ds(n * (2 * HB) + c * HB, HB)], sem_x)

    def stage_wait():
        pltpu.make_async_copy(xy_hbm.at[pl.ds(0, HB)],
                              xy_s.at[pl.ds(0, HB)], sem_x).wait()

    def stage0(o, _):
        stage_one(0, o)

        @pl.when(o >= 16)
        def _w():
            stage_wait()
        return 0

    lax.fori_loop(0, 2 * N, stage0, 0)

    def stage0_drain(i, _):
        stage_wait()
        return 0

    lax.fori_loop(0, 16, stage0_drain, 0)

    def half_body(h, _):
        def ib(n, _):
            base = n * (2 * HB)
            for h8 in range(HB // L):
                x16 = xy_s[pl.ds(base + h8 * L, L)]
                y16 = xy_s[pl.ds(base + HB + h8 * L, L)]
                plsc.store_scatter(idxbuf, [lane_base + (h8 * L * N + n)],
                                   x16 * 20 + y16)
            return 0

        lax.fori_loop(0, N, ib, 0)

        bt = wid * 4 + h // 2
        r0 = (bt * 128 + (h % 2) * HB) * N

        def gather(s, t):
            pltpu.async_copy(emb_hbm.at[idxbuf.at[pl.ds(s * K, K)]],
                             rbuf.at[t], gsems[t])

        for t in range(NBUF - 1):
            gather(t, t)

        def rounds(s4, _):
            for t in range(NBUF):
                s = s4 * NBUF + t
                @pl.when(h < HG - 1)
                def _stage():
                    @pl.when(s > 0)
                    def _sw():
                        for _ in range(SPR):
                            stage_wait()
                    for q in range(SPR):
                        stage_one(h + 1, s * SPR + q)
                u = (t + NBUF - 1) % NBUF
                if t == 0:
                    @pl.when(s4 > 0)
                    def _re0():
                        pltpu.make_async_copy(rbuf.at[u],
                                              out_hbm.at[pl.ds(0, K)],
                                              ssems[u]).wait()
                    gather(s + NBUF - 1, u)
                else:
                    @pl.when(s4 < STEPS // NBUF - 1)
                    def _re():
                        pltpu.make_async_copy(rbuf.at[u],
                                              out_hbm.at[pl.ds(0, K)],
                                              ssems[u]).wait()
                        gather(s + NBUF - 1, u)
                pltpu.make_async_copy(emb_hbm.at[pl.ds(0, K)],
                                      rbuf.at[t], gsems[t]).wait()
                pltpu.async_copy(rbuf.at[t],
                                 out_hbm.at[pl.ds(r0 + s * K, K)], ssems[t])
            return 0

        lax.fori_loop(0, STEPS // NBUF, rounds, 0)

        for t in range(NBUF):
            pltpu.make_async_copy(rbuf.at[t], out_hbm.at[pl.ds(0, K)],
                                  ssems[t]).wait()

        @pl.when(h < HG - 1)
        def _sdrain():
            for _ in range(SPR):
                stage_wait()
        return 0

    lax.fori_loop(0, HG, half_body, 0)


def kernel(xy, embedding):
    xyf = (xy.transpose(1, 0, 2)
             .reshape(N, 128, 128, 2)
             .transpose(0, 1, 3, 2)
             .reshape(-1))
    out = _sc_embed(xyf, embedding)
    return out.reshape(B, N, D).transpose(0, 2, 1)


# speedup vs baseline: 4.6052x; 4.6052x over previous
import functools

import jax
import jax.numpy as jnp
from jax import lax
from jax.experimental import pallas as pl
from jax.experimental.pallas import tpu as pltpu
from jax.experimental.pallas import tpu_sc as plsc

B = 16384
N = 200
D = 128
V = 400
L = 16
K = 128
NBUF = 4

_info = plsc.get_sparse_core_info()
NC, NS = _info.num_cores, _info.num_subcores
NW = NC * NS
B_PER_W = B // NW
HB = 64
HG = B_PER_W // HB
ROWS_H = HB * N
STEPS = ROWS_H // K
SPR = 4

_mesh = plsc.VectorSubcoreMesh(core_axis_name="c", subcore_axis_name="s")


@functools.partial(
    pl.kernel,
    mesh=_mesh,
    out_type=jax.ShapeDtypeStruct((B * N, D), jnp.float32),
    scratch_types=[
        pltpu.VMEM((N * 2 * HB,), jnp.int32),
        pltpu.VMEM((HB * N,), jnp.int32),
        pltpu.VMEM((NBUF, K, D), jnp.float32),
        pltpu.VMEM_SHARED((V, D), jnp.float32),
        pltpu.SemaphoreType.DMA,
        (pltpu.SemaphoreType.DMA,) * NBUF,
        (pltpu.SemaphoreType.DMA,) * NBUF,
    ],
    compiler_params=pltpu.CompilerParams(needs_layout_passes=False),
)
def _sc_embed(xy_hbm, emb_hbm, out_hbm, xy_s, idxbuf, rbuf, emb_sh,
              sem_x, gsems, ssems):
    wid = lax.axis_index("s") * NC + lax.axis_index("c")
    iota = jnp.arange(L, dtype=jnp.int32)
    lane_base = iota * N

    @pl.when(lax.axis_index("s") == 0)
    def _stage_table():
        pltpu.sync_copy(emb_hbm, emb_sh)

    plsc.subcore_barrier()

    def stage_one(h, o):
        bt = wid * 4 + h // 2
        bl0 = (h % 2) * HB
        n = o // 2
        c = o % 2
        pltpu.async_copy(
            xy_hbm.at[pl.ds(n * (128 * 256) + bt * 256 + c * 128 + bl0, HB)],
            xy_s.at[pl.ds(n * (2 * HB) + c * HB, HB)], sem_x)

    def stage_wait():
        pltpu.make_async_copy(xy_hbm.at[pl.ds(0, HB)],
                              xy_s.at[pl.ds(0, HB)], sem_x).wait()

    def stage0(o, _):
        stage_one(0, o)

        @pl.when(o >= 16)
        def _w():
            stage_wait()
        return 0

    lax.fori_loop(0, 2 * N, stage0, 0)

    def stage0_drain(i, _):
        stage_wait()
        return 0

    lax.fori_loop(0, 16, stage0_drain, 0)

    def half_body(h, _):
        def ib(n, _):
            base = n * (2 * HB)
            for h8 in range(HB // L):
                x16 = xy_s[pl.ds(base + h8 * L, L)]
                y16 = xy_s[pl.ds(base + HB + h8 * L, L)]
                plsc.store_scatter(idxbuf, [lane_base + (h8 * L * N + n)],
                                   x16 * 20 + y16)
            return 0

        lax.fori_loop(0, N, ib, 0)

        bt = wid * 4 + h // 2
        r0 = (bt * 128 + (h % 2) * HB) * N

        def gather(s, t):
            pltpu.async_copy(emb_sh.at[idxbuf.at[pl.ds(s * K, K)]],
                             rbuf.at[t], gsems[t])

        for t in range(NBUF - 1):
            gather(t, t)

        def rounds(s4, _):
            for t in range(NBUF):
                s = s4 * NBUF + t
                @pl.when(h < HG - 1)
                def _stage():
                    @pl.when(s > 0)
                    def _sw():
                        for _ in range(SPR):
                            stage_wait()
                    for q in range(SPR):
                        stage_one(h + 1, s * SPR + q)
                u = (t + NBUF - 1) % NBUF
                if t == 0:
                    @pl.when(s4 > 0)
                    def _re0():
                        pltpu.make_async_copy(rbuf.at[u],
                                              out_hbm.at[pl.ds(0, K)],
                                              ssems[u]).wait()
                    gather(s + NBUF - 1, u)
                else:
                    @pl.when(s4 < STEPS // NBUF - 1)
                    def _re():
                        pltpu.make_async_copy(rbuf.at[u],
                                              out_hbm.at[pl.ds(0, K)],
                                              ssems[u]).wait()
                        gather(s + NBUF - 1, u)
                pltpu.make_async_copy(emb_hbm.at[pl.ds(0, K)],
                                      rbuf.at[t], gsems[t]).wait()
                pltpu.async_copy(rbuf.at[t],
                                 out_hbm.at[pl.ds(r0 + s * K, K)], ssems[t])
            return 0

        lax.fori_loop(0, STEPS // NBUF, rounds, 0)

        for t in range(NBUF):
            pltpu.make_async_copy(rbuf.at[t], out_hbm.at[pl.ds(0, K)],
                                  ssems[t]).wait()

        @pl.when(h < HG - 1)
        def _sdrain():
            for _ in range(SPR):
                stage_wait()
        return 0

    lax.fori_loop(0, HG, half_body, 0)


def kernel(xy, embedding):
    xyf = (xy.transpose(1, 0, 2)
             .reshape(N, 128, 128, 2)
             .transpose(0, 1, 3, 2)
             .reshape(-1))
    out = _sc_embed(xyf, embedding)
    return out.reshape(B, N, D).transpose(0, 2, 1)


# speedup vs baseline: 4.8107x; 1.0446x over previous
import functools

import jax
import jax.numpy as jnp
from jax import lax
from jax.experimental import pallas as pl
from jax.experimental.pallas import tpu as pltpu
from jax.experimental.pallas import tpu_sc as plsc

B = 16384
N = 200
D = 128
V = 400
L = 16
K = 128
NBUF = 4

_info = plsc.get_sparse_core_info()
NC, NS = _info.num_cores, _info.num_subcores
NW = NC * NS
B_PER_W = B // NW
HB = 64
HG = B_PER_W // HB
ROWS_H = HB * N
STEPS = ROWS_H // K
SPR = 4

_mesh = plsc.VectorSubcoreMesh(core_axis_name="c", subcore_axis_name="s")


@functools.partial(
    pl.kernel,
    mesh=_mesh,
    out_type=jax.ShapeDtypeStruct((B * N, D), jnp.float32),
    scratch_types=[
        pltpu.VMEM((N * 2 * HB,), jnp.int32),
        pltpu.VMEM((2 * HB * N,), jnp.int32),
        pltpu.VMEM((NBUF, K, D), jnp.float32),
        pltpu.VMEM_SHARED((V, D), jnp.float32),
        pltpu.SemaphoreType.DMA,
        (pltpu.SemaphoreType.DMA,) * NBUF,
        (pltpu.SemaphoreType.DMA,) * NBUF,
    ],
    compiler_params=pltpu.CompilerParams(needs_layout_passes=False),
)
def _sc_embed(xy_hbm, emb_hbm, out_hbm, xy_s, idxbuf, rbuf, emb_sh,
              sem_x, gsems, ssems):
    wid = lax.axis_index("s") * NC + lax.axis_index("c")
    iota = jnp.arange(L, dtype=jnp.int32)
    lane_base = iota * N

    @pl.when(lax.axis_index("s") == 0)
    def _stage_table():
        pltpu.sync_copy(emb_hbm, emb_sh)

    plsc.subcore_barrier()

    def stage_one(h, o):
        bt = wid * 4 + h // 2
        bl0 = (h % 2) * HB
        n = o // 2
        c = o % 2
        pltpu.async_copy(
            xy_hbm.at[pl.ds(n * (128 * 256) + bt * 256 + c * 128 + bl0, HB)],
            xy_s.at[pl.ds(n * (2 * HB) + c * HB, HB)], sem_x)

    def stage_wait():
        pltpu.make_async_copy(xy_hbm.at[pl.ds(0, HB)],
                              xy_s.at[pl.ds(0, HB)], sem_x).wait()

    def stage0(o, _):
        stage_one(0, o)

        @pl.when(o >= 16)
        def _w():
            stage_wait()
        return 0

    lax.fori_loop(0, 2 * N, stage0, 0)

    def stage0_drain(i, _):
        stage_wait()
        return 0

    lax.fori_loop(0, 16, stage0_drain, 0)

    def build_n(n, dst_off):
        base = n * (2 * HB)
        for h8 in range(HB // L):
            x16 = xy_s[pl.ds(base + h8 * L, L)]
            y16 = xy_s[pl.ds(base + HB + h8 * L, L)]
            plsc.store_scatter(idxbuf,
                               [lane_base + (h8 * L * N + n) + dst_off],
                               x16 * 20 + y16)

    def ib0(n, _):
        build_n(n, 0)
        return 0

    lax.fori_loop(0, N, ib0, 0)

    def half_body(h, _):
        pb = h % 2
        g_off = pb * ROWS_H
        b_off = (1 - pb) * ROWS_H

        bt = wid * 4 + h // 2
        r0 = (bt * 128 + (h % 2) * HB) * N

        def gather(s, t):
            pltpu.async_copy(
                emb_sh.at[idxbuf.at[pl.ds(g_off + s * K, K)]],
                rbuf.at[t], gsems[t])

        for t in range(NBUF - 1):
            gather(t, t)

        def rounds(s4, _):
            for t in range(NBUF):
                s = s4 * NBUF + t
                @pl.when(h < HG - 1)
                def _stage():
                    @pl.when(s > 0)
                    def _sw():
                        for _ in range(SPR):
                            stage_wait()
                        build_n(2 * s - 2, b_off)
                        build_n(2 * s - 1, b_off)
                    for q in range(SPR):
                        stage_one(h + 1, s * SPR + q)
                u = (t + NBUF - 1) % NBUF
                if t == 0:
                    @pl.when(s4 > 0)
                    def _re0():
                        pltpu.make_async_copy(rbuf.at[u],
                                              out_hbm.at[pl.ds(0, K)],
                                              ssems[u]).wait()
                    gather(s + NBUF - 1, u)
                else:
                    @pl.when(s4 < STEPS // NBUF - 1)
                    def _re():
                        pltpu.make_async_copy(rbuf.at[u],
                                              out_hbm.at[pl.ds(0, K)],
                                              ssems[u]).wait()
                        gather(s + NBUF - 1, u)
                pltpu.make_async_copy(emb_hbm.at[pl.ds(0, K)],
                                      rbuf.at[t], gsems[t]).wait()
                pltpu.async_copy(rbuf.at[t],
                                 out_hbm.at[pl.ds(r0 + s * K, K)], ssems[t])
            return 0

        lax.fori_loop(0, STEPS // NBUF, rounds, 0)

        for t in range(NBUF):
            pltpu.make_async_copy(rbuf.at[t], out_hbm.at[pl.ds(0, K)],
                                  ssems[t]).wait()

        @pl.when(h < HG - 1)
        def _sdrain():
            for _ in range(SPR):
                stage_wait()
            build_n(N - 2, b_off)
            build_n(N - 1, b_off)
        return 0

    lax.fori_loop(0, HG, half_body, 0)


def kernel(xy, embedding):
    xyf = (xy.transpose(1, 0, 2)
             .reshape(N, 128, 128, 2)
             .transpose(0, 1, 3, 2)
             .reshape(-1))
    out = _sc_embed(xyf, embedding)
    return out.reshape(B, N, D).transpose(0, 2, 1)
